# baseline (speedup 1.0000x reference)
import jax
import jax.numpy as jnp
from jax.experimental import pallas as pl
from jax.experimental.pallas import tpu as pltpu

EPS = 1e-7
ALPHA = 0.2
_MAXNORM = 1.0 - 1e-5


def _prologue_body(x_ref, w_ref, a1_ref, a2_ref,
                   whx_ref, ut_ref, upt_ref, v_ref, vp_ref):
    x = x_ref[...]
    wh0 = jax.lax.dot_general(x, w_ref[...], (((1,), (0,)), ((), ())),
                              preferred_element_type=jnp.float32)
    ones_row = jnp.ones((1, x.shape[1]), jnp.float32)
    nrm2t = jax.lax.dot_general(ones_row, x * x, (((1,), (1,)), ((), ())),
                                preferred_element_type=jnp.float32,
                                precision=jax.lax.Precision.HIGHEST)
    nrmt = jnp.maximum(jnp.sqrt(nrm2t), EPS)
    t = jnp.clip(nrmt, -1.0 + 1e-5, 1.0 - 1e-5)
    artt = 0.5 * jnp.log((1.0 + t) / (1.0 - t))
    fact = artt / nrmt
    f1t = fact * jax.lax.dot_general(a1_ref[...], wh0, (((1,), (1,)), ((), ())),
                                     preferred_element_type=jnp.float32,
                                     precision=jax.lax.Precision.HIGHEST)
    f2t = fact * jax.lax.dot_general(a2_ref[...], wh0, (((1,), (1,)), ((), ())),
                                     preferred_element_type=jnp.float32,
                                     precision=jax.lax.Precision.HIGHEST)
    m = f1t + jnp.max(f2t)
    m = jnp.where(m >= 0, m, ALPHA * m)
    ut_ref[...] = jnp.exp(f1t - m)
    upt_ref[...] = jnp.exp(ALPHA * f1t - m)
    v_ref[...] = (jnp.exp(f2t) * fact).astype(jnp.bfloat16)
    vp_ref[...] = (jnp.exp(ALPHA * f2t) * fact).astype(jnp.bfloat16)
    whx_ref[:, :-1] = wh0.astype(jnp.bfloat16)
    whx_ref[:, -1:] = (nrmt / artt).astype(jnp.bfloat16).T


def _attn_body(whx_ref, ut_ref, upt_ref, v_ref, vp_ref, adj_ref, b_ref,
               out_ref):
    u = ut_ref[...].T.astype(jnp.bfloat16)
    up = upt_ref[...].T.astype(jnp.bfloat16)
    qadj = adj_ref[...].astype(jnp.bfloat16)
    p = jnp.maximum(u * v_ref[...], up * vp_ref[...])
    p = jnp.where(qadj > 0, p, jnp.bfloat16(0.0))
    acc = jax.lax.dot_general(p, whx_ref[...], (((1,), (0,)), ((), ())),
                              preferred_element_type=jnp.float32)
    hp = acc[:, :-1] / acc[:, -1:] + b_ref[...]
    out = jnp.where(hp > 0, hp, jnp.exp(jnp.minimum(hp, 0.0)) - 1.0)
    onrm = jnp.maximum(jnp.sqrt(jnp.sum(out * out, axis=1, keepdims=True)), EPS)
    res = out * (jnp.tanh(onrm) / onrm)
    rn = jnp.maximum(jnp.sqrt(jnp.sum(res * res, axis=1, keepdims=True)), EPS)
    out_ref[...] = jnp.where(rn > _MAXNORM, res * (_MAXNORM / rn), res)


def kernel(x, adj, W, a, b):
    N, D = x.shape
    C = W.shape[1]
    a1 = a[:C].reshape(1, C)
    a2 = a[C:].reshape(1, C)
    b2 = b.reshape(1, C)

    whx, ut, upt, v, vp = pl.pallas_call(
        _prologue_body,
        out_shape=(
            jax.ShapeDtypeStruct((N, C + 1), jnp.bfloat16),
            jax.ShapeDtypeStruct((1, N), jnp.float32),
            jax.ShapeDtypeStruct((1, N), jnp.float32),
            jax.ShapeDtypeStruct((1, N), jnp.bfloat16),
            jax.ShapeDtypeStruct((1, N), jnp.bfloat16),
        ),
    )(x, W, a1, a2)

    BR = 512
    grid = pl.cdiv(N, BR)
    out = pl.pallas_call(
        _attn_body,
        grid=(grid,),
        in_specs=[
            pl.BlockSpec((N, C + 1), lambda i: (0, 0)),
            pl.BlockSpec((1, BR), lambda i: (0, i)),
            pl.BlockSpec((1, BR), lambda i: (0, i)),
            pl.BlockSpec((1, N), lambda i: (0, 0)),
            pl.BlockSpec((1, N), lambda i: (0, 0)),
            pl.BlockSpec((BR, N), lambda i: (i, 0)),
            pl.BlockSpec((1, C), lambda i: (0, 0)),
        ],
        out_specs=pl.BlockSpec((BR, C), lambda i: (i, 0)),
        out_shape=jax.ShapeDtypeStruct((N, C), jnp.float32),
        compiler_params=pltpu.CompilerParams(
            vmem_limit_bytes=64 * 1024 * 1024),
    )(whx, ut, upt, v, vp, adj, b2)
    return out


# speedup vs baseline: 1.0358x; 1.0358x over previous
import jax
import jax.numpy as jnp
from jax.experimental import pallas as pl
from jax.experimental.pallas import tpu as pltpu

EPS = 1e-7
ALPHA = 0.2
_MAXNORM = 1.0 - 1e-5


def _prologue_body(x_ref, w_ref, a1_ref, a2_ref,
                   whx_ref, ut_ref, upt_ref, v_ref, vp_ref):
    x = x_ref[...]
    wh0 = jax.lax.dot_general(x, w_ref[...], (((1,), (0,)), ((), ())),
                              preferred_element_type=jnp.float32)
    nrm2t = jnp.sum(x * x, axis=1, keepdims=True).T
    nrmt = jnp.maximum(jnp.sqrt(nrm2t), EPS)
    t = jnp.clip(nrmt, -1.0 + 1e-5, 1.0 - 1e-5)
    artt = 0.5 * jnp.log((1.0 + t) / (1.0 - t))
    fact = artt / nrmt
    f1t = fact * jax.lax.dot_general(a1_ref[...], wh0, (((1,), (1,)), ((), ())),
                                     preferred_element_type=jnp.float32)
    f2t = fact * jax.lax.dot_general(a2_ref[...], wh0, (((1,), (1,)), ((), ())),
                                     preferred_element_type=jnp.float32)
    m = f1t + jnp.max(f2t)
    m = jnp.where(m >= 0, m, ALPHA * m)
    ut_ref[...] = jnp.exp(f1t - m)
    upt_ref[...] = jnp.exp(ALPHA * f1t - m)
    v_ref[...] = (jnp.exp(f2t) * fact).astype(jnp.bfloat16)
    vp_ref[...] = (jnp.exp(ALPHA * f2t) * fact).astype(jnp.bfloat16)
    whx_ref[:, :-1] = wh0.astype(jnp.bfloat16)
    whx_ref[:, -1:] = (nrmt / artt).astype(jnp.bfloat16).T


def _attn_body(whx_ref, ut_ref, upt_ref, v_ref, vp_ref, adj_ref, b_ref,
               out_ref):
    u = ut_ref[...].T.astype(jnp.bfloat16)
    up = upt_ref[...].T.astype(jnp.bfloat16)
    qadj = adj_ref[...].astype(jnp.bfloat16)
    p = jnp.maximum(u * v_ref[...], up * vp_ref[...])
    p = jnp.where(qadj > 0, p, jnp.bfloat16(0.0))
    acc = jax.lax.dot_general(p, whx_ref[...], (((1,), (0,)), ((), ())),
                              preferred_element_type=jnp.float32)
    hp = acc[:, :-1] / acc[:, -1:] + b_ref[...]
    out = jnp.where(hp > 0, hp, jnp.exp(jnp.minimum(hp, 0.0)) - 1.0)
    onrm = jnp.maximum(jnp.sqrt(jnp.sum(out * out, axis=1, keepdims=True)), EPS)
    res = out * (jnp.tanh(onrm) / onrm)
    rn = jnp.maximum(jnp.sqrt(jnp.sum(res * res, axis=1, keepdims=True)), EPS)
    out_ref[...] = jnp.where(rn > _MAXNORM, res * (_MAXNORM / rn), res)


def kernel(x, adj, W, a, b):
    N, D = x.shape
    C = W.shape[1]
    a1 = a[:C].reshape(1, C)
    a2 = a[C:].reshape(1, C)
    b2 = b.reshape(1, C)

    whx, ut, upt, v, vp = pl.pallas_call(
        _prologue_body,
        out_shape=(
            jax.ShapeDtypeStruct((N, C + 1), jnp.bfloat16),
            jax.ShapeDtypeStruct((1, N), jnp.float32),
            jax.ShapeDtypeStruct((1, N), jnp.float32),
            jax.ShapeDtypeStruct((1, N), jnp.bfloat16),
            jax.ShapeDtypeStruct((1, N), jnp.bfloat16),
        ),
    )(x, W, a1, a2)

    BR = 512
    grid = pl.cdiv(N, BR)
    out = pl.pallas_call(
        _attn_body,
        grid=(grid,),
        in_specs=[
            pl.BlockSpec((N, C + 1), lambda i: (0, 0)),
            pl.BlockSpec((1, BR), lambda i: (0, i)),
            pl.BlockSpec((1, BR), lambda i: (0, i)),
            pl.BlockSpec((1, N), lambda i: (0, 0)),
            pl.BlockSpec((1, N), lambda i: (0, 0)),
            pl.BlockSpec((BR, N), lambda i: (i, 0)),
            pl.BlockSpec((1, C), lambda i: (0, 0)),
        ],
        out_specs=pl.BlockSpec((BR, C), lambda i: (i, 0)),
        out_shape=jax.ShapeDtypeStruct((N, C), jnp.float32),
        compiler_params=pltpu.CompilerParams(
            vmem_limit_bytes=64 * 1024 * 1024),
    )(whx, ut, upt, v, vp, adj, b2)
    return out


# speedup vs baseline: 1.0976x; 1.0597x over previous
import jax
import jax.numpy as jnp
from jax.experimental import pallas as pl
from jax.experimental.pallas import tpu as pltpu

EPS = 1e-7
ALPHA = 0.2
_MAXNORM = 1.0 - 1e-5


def _fused_body(x_ref, w_ref, a1_ref, a2_ref, adj_ref, b_ref, out_ref,
                whx_s, ut_s, upt_s, v_s, vp_s):
    pid = pl.program_id(0)

    @pl.when(pid == 0)
    def _prologue():
        x = x_ref[...]
        wh0 = jax.lax.dot_general(x, w_ref[...], (((1,), (0,)), ((), ())),
                                  preferred_element_type=jnp.float32)
        nrm2t = jnp.sum(x * x, axis=1, keepdims=True).T
        nrmt = jnp.maximum(jnp.sqrt(nrm2t), EPS)
        t = jnp.clip(nrmt, -1.0 + 1e-5, 1.0 - 1e-5)
        artt = 0.5 * jnp.log((1.0 + t) / (1.0 - t))
        fact = artt / nrmt
        f1t = fact * jax.lax.dot_general(
            a1_ref[...], wh0, (((1,), (1,)), ((), ())),
            preferred_element_type=jnp.float32)
        f2t = fact * jax.lax.dot_general(
            a2_ref[...], wh0, (((1,), (1,)), ((), ())),
            preferred_element_type=jnp.float32)
        m = f1t + jnp.max(f2t)
        m = jnp.where(m >= 0, m, ALPHA * m)
        ut_s[:, :f1t.shape[1]] = jnp.exp(f1t - m)
        upt_s[:, :f1t.shape[1]] = jnp.exp(ALPHA * f1t - m)
        ut_s[:, f1t.shape[1]:] = jnp.zeros_like(ut_s[:, f1t.shape[1]:])
        upt_s[:, f1t.shape[1]:] = jnp.zeros_like(upt_s[:, f1t.shape[1]:])
        v_s[...] = (jnp.exp(f2t) * fact).astype(jnp.bfloat16)
        vp_s[...] = (jnp.exp(ALPHA * f2t) * fact).astype(jnp.bfloat16)
        whx_s[:, :-1] = wh0.astype(jnp.bfloat16)
        whx_s[:, -1:] = (nrmt / artt).astype(jnp.bfloat16).T

    br = adj_ref.shape[0]
    base = pid * br
    u = ut_s[:, pl.ds(base, br)].T.astype(jnp.bfloat16)
    up = upt_s[:, pl.ds(base, br)].T.astype(jnp.bfloat16)
    qadj = adj_ref[...].astype(jnp.bfloat16)
    p = jnp.maximum(u * v_s[...], up * vp_s[...])
    p = jnp.where(qadj > 0, p, jnp.bfloat16(0.0))
    acc = jax.lax.dot_general(p, whx_s[...], (((1,), (0,)), ((), ())),
                              preferred_element_type=jnp.float32)
    hp = acc[:, :-1] / acc[:, -1:] + b_ref[...]
    out = jnp.where(hp > 0, hp, jnp.exp(jnp.minimum(hp, 0.0)) - 1.0)
    onrm = jnp.maximum(jnp.sqrt(jnp.sum(out * out, axis=1, keepdims=True)), EPS)
    res = out * (jnp.tanh(onrm) / onrm)
    rn = jnp.maximum(jnp.sqrt(jnp.sum(res * res, axis=1, keepdims=True)), EPS)
    out_ref[...] = jnp.where(rn > _MAXNORM, res * (_MAXNORM / rn), res)


def kernel(x, adj, W, a, b):
    N, D = x.shape
    C = W.shape[1]
    a1 = a[:C].reshape(1, C)
    a2 = a[C:].reshape(1, C)
    b2 = b.reshape(1, C)

    BR = 512
    grid = pl.cdiv(N, BR)
    out = pl.pallas_call(
        _fused_body,
        grid=(grid,),
        in_specs=[
            pl.BlockSpec((N, D), lambda i: (0, 0)),
            pl.BlockSpec((D, C), lambda i: (0, 0)),
            pl.BlockSpec((1, C), lambda i: (0, 0)),
            pl.BlockSpec((1, C), lambda i: (0, 0)),
            pl.BlockSpec((BR, N), lambda i: (i, 0)),
            pl.BlockSpec((1, C), lambda i: (0, 0)),
        ],
        out_specs=pl.BlockSpec((BR, C), lambda i: (i, 0)),
        out_shape=jax.ShapeDtypeStruct((N, C), jnp.float32),
        scratch_shapes=[
            pltpu.VMEM((N, C + 1), jnp.bfloat16),
            pltpu.VMEM((1, grid * BR), jnp.float32),
            pltpu.VMEM((1, grid * BR), jnp.float32),
            pltpu.VMEM((1, N), jnp.bfloat16),
            pltpu.VMEM((1, N), jnp.bfloat16),
        ],
        compiler_params=pltpu.CompilerParams(
            vmem_limit_bytes=64 * 1024 * 1024),
    )(x, W, a1, a2, adj, b2)
    return out
